# baseline (speedup 1.0000x reference)
import functools

import jax
import jax.numpy as jnp
from jax import lax
from jax.experimental import pallas as pl
from jax.experimental.pallas import tpu as pltpu
from jax.experimental.pallas import tpu_sc as plsc

_IM_FE_RATIO = 4.0
_PP = 2


@functools.lru_cache(maxsize=None)
def _build_sc_kernel(B, C, H, W, N):
    info = plsc.get_sparse_core_info()
    NC, NS, L = info.num_cores, info.num_subcores, info.num_lanes
    NW = NC * NS
    P = B * C
    assert P % NW == 0
    planes_per_w = P // NW
    assert C % planes_per_w == 0
    assert N % L == 0
    n_groups = planes_per_w // _PP
    assert n_groups * _PP == planes_per_w and n_groups % 2 == 0

    mesh = plsc.VectorSubcoreMesh(core_axis_name="c", subcore_axis_name="s")

    @functools.partial(
        pl.kernel,
        out_type=jax.ShapeDtypeStruct((P, N), jnp.float32),
        mesh=mesh,
        compiler_params=pltpu.CompilerParams(needs_layout_passes=False),
        scratch_types=[
            pltpu.VMEM((N,), jnp.float32),
            pltpu.VMEM((N,), jnp.float32),
            pltpu.VMEM((N,), jnp.int32),
            pltpu.VMEM((N,), jnp.int32),
            pltpu.VMEM((_PP, H, W), jnp.float32),
            pltpu.VMEM((_PP, H, W), jnp.float32),
            pltpu.VMEM((_PP * N,), jnp.float32),
            pltpu.VMEM((_PP * N,), jnp.float32),
            pltpu.SemaphoreType.DMA,
            pltpu.SemaphoreType.DMA,
            pltpu.SemaphoreType.DMA,
            pltpu.SemaphoreType.DMA,
        ],
    )
    def sc_kernel(feat_hbm, kpx_hbm, kpy_hbm, out_hbm,
                  kpx_v, kpy_v, idx_v, cau_v, buf_a, buf_b,
                  obuf_a, obuf_b, sem_a, sem_b, sem_oa, sem_ob):
        cid = lax.axis_index("c")
        sid = lax.axis_index("s")
        wid = sid * NC + cid
        p_lo = wid * planes_per_w
        b = p_lo // C
        c_lo = p_lo - b * C

        pltpu.sync_copy(kpx_hbm.at[b], kpx_v)
        pltpu.sync_copy(kpy_hbm.at[b], kpy_v)

        def start_group(g, buf, sem):
            pltpu.async_copy(
                feat_hbm.at[b, pl.ds(c_lo + g * _PP, _PP)], buf, sem)

        def wait_group(buf, sem):
            pltpu.make_async_copy(
                feat_hbm.at[b, pl.ds(c_lo, _PP)], buf, sem).wait()

        def start_out(g, obuf, sem):
            for j in range(_PP):
                pltpu.async_copy(obuf.at[pl.ds(j * N, N)],
                                 out_hbm.at[p_lo + g * _PP + j], sem)

        def wait_out(obuf, sem):
            for j in range(_PP):
                pltpu.make_async_copy(obuf.at[pl.ds(j * N, N)],
                                      out_hbm.at[p_lo + j], sem).wait()

        start_group(0, buf_a, sem_a)

        inv_ratio = jnp.float32(1.0 / _IM_FE_RATIO)

        @plsc.parallel_loop(0, N, step=L, unroll=2)
        def precompute(kb):
            x = kpx_v[pl.ds(kb, L)]
            y = kpy_v[pl.ds(kb, L)]
            code = ((x > 1e-10).astype(jnp.int32)
                    + (y > 1e-10).astype(jnp.int32))
            mask = code.astype(jnp.float32) * jnp.float32(0.5)
            xs = x * inv_ratio
            ys = y * inv_ratio
            fxi = jnp.maximum(xs.astype(jnp.int32), 0)
            fyi = jnp.maximum(ys.astype(jnp.int32), 0)
            fxf = fxi.astype(jnp.float32)
            fyf = fyi.astype(jnp.float32)
            ux = xs - fxf
            uy = ys - fyf
            cxi = jnp.minimum(fxi + (xs > fxf).astype(jnp.int32), W - 1)
            cyi = jnp.minimum(fyi + (ys > fyf).astype(jnp.int32), H - 1)
            idx_v[pl.ds(kb, L)] = (fyi | (cyi << 7) | (fxi << 14)
                                   | (cxi << 21) | (code << 28))
            cau_v[pl.ds(kb, L)] = plsc.bitcast(
                plsc.pack(ux * mask, uy, format=plsc.PackFormat.INTERLEAVED),
                jnp.int32)

        def compute_group(buf, obuf):
            @plsc.parallel_loop(0, N, step=L, unroll=8)
            def chunk(kb):
                pc = idx_v[pl.ds(kb, L)]
                m7 = jnp.int32(127)
                fy = pc & m7
                cy = (pc >> 7) & m7
                fx = (pc >> 14) & m7
                cx = (pc >> 21) & m7
                av, uy = plsc.unpack(
                    plsc.bitcast(cau_v[pl.ds(kb, L)], jnp.bfloat16),
                    format=plsc.PackFormat.INTERLEAVED)
                m = (pc >> 28).astype(jnp.float32) * jnp.float32(0.5)
                lxm = m - av
                ly = jnp.float32(1.0) - uy
                for j in range(_PP):
                    pj = buf.at[j]
                    r0 = (plsc.load_gather(pj, [fy, fx]) * lxm
                          + plsc.load_gather(pj, [fy, cx]) * av)
                    r1 = (plsc.load_gather(pj, [cy, fx]) * lxm
                          + plsc.load_gather(pj, [cy, cx]) * av)
                    obuf[pl.ds(j * N + kb, L)] = r0 * ly + r1 * uy

        def pair_loop(i, carry):
            g0 = 2 * i
            wait_group(buf_a, sem_a)
            start_group(g0 + 1, buf_b, sem_b)

            @pl.when(i > 0)
            def _():
                wait_out(obuf_a, sem_oa)

            compute_group(buf_a, obuf_a)
            start_out(g0, obuf_a, sem_oa)

            wait_group(buf_b, sem_b)

            @pl.when(i < n_groups // 2 - 1)
            def _():
                start_group(g0 + 2, buf_a, sem_a)

            @pl.when(i > 0)
            def _():
                wait_out(obuf_b, sem_ob)

            compute_group(buf_b, obuf_b)
            start_out(g0 + 1, obuf_b, sem_ob)
            return carry

        lax.fori_loop(0, n_groups // 2, pair_loop, 0)
        wait_out(obuf_a, sem_oa)
        wait_out(obuf_b, sem_ob)

    return sc_kernel


def kernel(feature, keypoints):
    B, C, H, W = feature.shape
    N = keypoints.shape[1]
    kpx = keypoints[:, :, 0]
    kpy = keypoints[:, :, 1]
    out = _build_sc_kernel(B, C, H, W, N)(feature, kpx, kpy)
    return out.reshape(B, C, N)


# speedup vs baseline: 1.5394x; 1.5394x over previous
import functools

import jax
import jax.numpy as jnp
from jax import lax
from jax.experimental import pallas as pl
from jax.experimental.pallas import tpu as pltpu
from jax.experimental.pallas import tpu_sc as plsc

_IM_FE_RATIO = 4.0
_PP = 2


@functools.lru_cache(maxsize=None)
def _build_sc_kernel(B, C, H, W, N):
    info = plsc.get_sparse_core_info()
    NC, NS, L = info.num_cores, info.num_subcores, info.num_lanes
    NW = NC * NS
    P = B * C
    assert P % NW == 0
    planes_per_w = P // NW
    assert C % planes_per_w == 0
    assert N % L == 0
    n_groups = planes_per_w // _PP
    assert n_groups * _PP == planes_per_w and n_groups % 2 == 0

    mesh = plsc.VectorSubcoreMesh(core_axis_name="c", subcore_axis_name="s")

    @functools.partial(
        pl.kernel,
        out_type=jax.ShapeDtypeStruct((P, N), jnp.float32),
        mesh=mesh,
        compiler_params=pltpu.CompilerParams(needs_layout_passes=False),
        scratch_types=[
            pltpu.VMEM((N,), jnp.float32),
            pltpu.VMEM((N,), jnp.float32),
            pltpu.VMEM((N,), jnp.int32),
            pltpu.VMEM((N,), jnp.int32),
            pltpu.VMEM((_PP, H, W), jnp.float32),
            pltpu.VMEM((_PP, H, W), jnp.float32),
            pltpu.VMEM((_PP * N,), jnp.float32),
            pltpu.VMEM((_PP * N,), jnp.float32),
            pltpu.SemaphoreType.DMA,
            pltpu.SemaphoreType.DMA,
            pltpu.SemaphoreType.DMA,
            pltpu.SemaphoreType.DMA,
        ],
    )
    def sc_kernel(feat_hbm, kpx_hbm, kpy_hbm, out_hbm,
                  kpx_v, kpy_v, idx_v, cau_v, buf_a, buf_b,
                  obuf_a, obuf_b, sem_a, sem_b, sem_oa, sem_ob):
        cid = lax.axis_index("c")
        sid = lax.axis_index("s")
        wid = sid * NC + cid
        p_lo = wid * planes_per_w
        b = p_lo // C
        c_lo = p_lo - b * C

        pltpu.sync_copy(kpx_hbm.at[b], kpx_v)
        pltpu.sync_copy(kpy_hbm.at[b], kpy_v)

        def start_group(g, buf, sem):
            pltpu.async_copy(
                feat_hbm.at[b, pl.ds(c_lo + g * _PP, _PP)], buf, sem)

        def wait_group(buf, sem):
            pltpu.make_async_copy(
                feat_hbm.at[b, pl.ds(c_lo, _PP)], buf, sem).wait()

        def start_out(g, obuf, sem):
            for j in range(_PP):
                pltpu.async_copy(obuf.at[pl.ds(j * N, N)],
                                 out_hbm.at[p_lo + g * _PP + j], sem)

        def wait_out(obuf, sem):
            for j in range(_PP):
                pltpu.make_async_copy(obuf.at[pl.ds(j * N, N)],
                                      out_hbm.at[p_lo + j], sem).wait()

        start_group(0, buf_a, sem_a)

        inv_ratio = jnp.float32(1.0 / _IM_FE_RATIO)

        @plsc.parallel_loop(0, N, step=L, unroll=2)
        def precompute(kb):
            x = kpx_v[pl.ds(kb, L)]
            y = kpy_v[pl.ds(kb, L)]
            code = ((x > 1e-10).astype(jnp.int32)
                    + (y > 1e-10).astype(jnp.int32))
            mask = code.astype(jnp.float32) * jnp.float32(0.5)
            xs = x * inv_ratio
            ys = y * inv_ratio
            fxi = jnp.maximum(xs.astype(jnp.int32), 0)
            fyi = jnp.maximum(ys.astype(jnp.int32), 0)
            fxf = fxi.astype(jnp.float32)
            fyf = fyi.astype(jnp.float32)
            ux = xs - fxf
            uy = ys - fyf
            cxi = jnp.minimum(fxi + (xs > fxf).astype(jnp.int32), W - 1)
            cyi = jnp.minimum(fyi + (ys > fyf).astype(jnp.int32), H - 1)
            idx_v[pl.ds(kb, L)] = (fyi | (cyi << 7) | (fxi << 14)
                                   | (cxi << 21) | (code << 28))
            cau_v[pl.ds(kb, L)] = plsc.bitcast(
                plsc.pack(ux * mask, uy, format=plsc.PackFormat.INTERLEAVED),
                jnp.int32)

        def compute_group(buf, obuf):
            @plsc.parallel_loop(0, N, step=L, unroll=4)
            def chunk(kb):
                pc = idx_v[pl.ds(kb, L)]
                m7 = jnp.int32(127)
                fy = pc & m7
                cy = (pc >> 7) & m7
                fx = (pc >> 14) & m7
                cx = (pc >> 21) & m7
                av, uy = plsc.unpack(
                    plsc.bitcast(cau_v[pl.ds(kb, L)], jnp.bfloat16),
                    format=plsc.PackFormat.INTERLEAVED)
                m = (pc >> 28).astype(jnp.float32) * jnp.float32(0.5)
                lxm = m - av
                ly = jnp.float32(1.0) - uy
                for j in range(_PP):
                    pj = buf.at[j]
                    r0 = (plsc.load_gather(pj, [fy, fx]) * lxm
                          + plsc.load_gather(pj, [fy, cx]) * av)
                    r1 = (plsc.load_gather(pj, [cy, fx]) * lxm
                          + plsc.load_gather(pj, [cy, cx]) * av)
                    obuf[pl.ds(j * N + kb, L)] = r0 * ly + r1 * uy

        def pair_loop(i, carry):
            g0 = 2 * i
            wait_group(buf_a, sem_a)
            start_group(g0 + 1, buf_b, sem_b)

            @pl.when(i > 0)
            def _():
                wait_out(obuf_a, sem_oa)

            compute_group(buf_a, obuf_a)
            start_out(g0, obuf_a, sem_oa)

            wait_group(buf_b, sem_b)

            @pl.when(i < n_groups // 2 - 1)
            def _():
                start_group(g0 + 2, buf_a, sem_a)

            @pl.when(i > 0)
            def _():
                wait_out(obuf_b, sem_ob)

            compute_group(buf_b, obuf_b)
            start_out(g0 + 1, obuf_b, sem_ob)
            return carry

        lax.fori_loop(0, n_groups // 2, pair_loop, 0)
        wait_out(obuf_a, sem_oa)
        wait_out(obuf_b, sem_ob)

    return sc_kernel


def kernel(feature, keypoints):
    B, C, H, W = feature.shape
    N = keypoints.shape[1]
    kpx = keypoints[:, :, 0]
    kpy = keypoints[:, :, 1]
    out = _build_sc_kernel(B, C, H, W, N)(feature, kpx, kpy)
    return out.reshape(B, C, N)


# speedup vs baseline: 1.6782x; 1.0902x over previous
import functools

import jax
import jax.numpy as jnp
from jax import lax
from jax.experimental import pallas as pl
from jax.experimental.pallas import tpu as pltpu
from jax.experimental.pallas import tpu_sc as plsc

_IM_FE_RATIO = 4.0
_PP = 2


@functools.lru_cache(maxsize=None)
def _build_sc_kernel(B, C, H, W, N):
    info = plsc.get_sparse_core_info()
    NC, NS, L = info.num_cores, info.num_subcores, info.num_lanes
    NW = NC * NS
    P = B * C
    assert P % NW == 0
    planes_per_w = P // NW
    assert C % planes_per_w == 0
    assert N % L == 0
    n_groups = planes_per_w // _PP
    assert n_groups * _PP == planes_per_w and n_groups % 2 == 0

    mesh = plsc.VectorSubcoreMesh(core_axis_name="c", subcore_axis_name="s")

    @functools.partial(
        pl.kernel,
        out_type=jax.ShapeDtypeStruct((P, N), jnp.float32),
        mesh=mesh,
        compiler_params=pltpu.CompilerParams(needs_layout_passes=False),
        scratch_types=[
            pltpu.VMEM((N,), jnp.float32),
            pltpu.VMEM((N,), jnp.float32),
            pltpu.VMEM((N,), jnp.int32),
            pltpu.VMEM((N,), jnp.float32),
            pltpu.VMEM((N,), jnp.float32),
            pltpu.VMEM((_PP, H, W), jnp.float32),
            pltpu.VMEM((_PP, H, W), jnp.float32),
            pltpu.VMEM((_PP * N,), jnp.float32),
            pltpu.VMEM((_PP * N,), jnp.float32),
            pltpu.SemaphoreType.DMA,
            pltpu.SemaphoreType.DMA,
            pltpu.SemaphoreType.DMA,
            pltpu.SemaphoreType.DMA,
        ],
    )
    def sc_kernel(feat_hbm, kpx_hbm, kpy_hbm, out_hbm,
                  kpx_v, kpy_v, idx_v, ca_v, cu_v, buf_a, buf_b,
                  obuf_a, obuf_b, sem_a, sem_b, sem_oa, sem_ob):
        cid = lax.axis_index("c")
        sid = lax.axis_index("s")
        wid = sid * NC + cid
        p_lo = wid * planes_per_w
        b = p_lo // C
        c_lo = p_lo - b * C

        pltpu.sync_copy(kpx_hbm.at[b], kpx_v)
        pltpu.sync_copy(kpy_hbm.at[b], kpy_v)

        def start_group(g, buf, sem):
            pltpu.async_copy(
                feat_hbm.at[b, pl.ds(c_lo + g * _PP, _PP)], buf, sem)

        def wait_group(buf, sem):
            pltpu.make_async_copy(
                feat_hbm.at[b, pl.ds(c_lo, _PP)], buf, sem).wait()

        def start_out(g, obuf, sem):
            for j in range(_PP):
                pltpu.async_copy(obuf.at[pl.ds(j * N, N)],
                                 out_hbm.at[p_lo + g * _PP + j], sem)

        def wait_out(obuf, sem):
            for j in range(_PP):
                pltpu.make_async_copy(obuf.at[pl.ds(j * N, N)],
                                      out_hbm.at[p_lo + j], sem).wait()

        start_group(0, buf_a, sem_a)

        inv_ratio = jnp.float32(1.0 / _IM_FE_RATIO)

        @plsc.parallel_loop(0, N, step=L, unroll=2)
        def precompute(kb):
            x = kpx_v[pl.ds(kb, L)]
            y = kpy_v[pl.ds(kb, L)]
            code = ((x > 1e-10).astype(jnp.int32)
                    + (y > 1e-10).astype(jnp.int32))
            mask = code.astype(jnp.float32) * jnp.float32(0.5)
            xs = x * inv_ratio
            ys = y * inv_ratio
            fxi = jnp.maximum(xs.astype(jnp.int32), 0)
            fyi = jnp.maximum(ys.astype(jnp.int32), 0)
            fxf = fxi.astype(jnp.float32)
            fyf = fyi.astype(jnp.float32)
            ux = xs - fxf
            uy = ys - fyf
            cxi = jnp.minimum(fxi + (xs > fxf).astype(jnp.int32), W - 1)
            cyi = jnp.minimum(fyi + (ys > fyf).astype(jnp.int32), H - 1)
            idx_v[pl.ds(kb, L)] = (fyi | (cyi << 7) | (fxi << 14)
                                   | (cxi << 21) | (code << 28))
            ca_v[pl.ds(kb, L)] = ux * mask
            cu_v[pl.ds(kb, L)] = uy

        def compute_group(buf, obuf):
            @plsc.parallel_loop(0, N, step=L, unroll=4)
            def chunk(kb):
                pc = idx_v[pl.ds(kb, L)]
                m7 = jnp.int32(127)
                fy = pc & m7
                cy = (pc >> 7) & m7
                fx = (pc >> 14) & m7
                cx = (pc >> 21) & m7
                av = ca_v[pl.ds(kb, L)]
                uy = cu_v[pl.ds(kb, L)]
                m = (pc >> 28).astype(jnp.float32) * jnp.float32(0.5)
                lxm = m - av
                ly = jnp.float32(1.0) - uy
                for j in range(_PP):
                    pj = buf.at[j]
                    r0 = (plsc.load_gather(pj, [fy, fx]) * lxm
                          + plsc.load_gather(pj, [fy, cx]) * av)
                    r1 = (plsc.load_gather(pj, [cy, fx]) * lxm
                          + plsc.load_gather(pj, [cy, cx]) * av)
                    obuf[pl.ds(j * N + kb, L)] = r0 * ly + r1 * uy

        def pair_loop(i, carry):
            g0 = 2 * i
            wait_group(buf_a, sem_a)
            start_group(g0 + 1, buf_b, sem_b)

            @pl.when(i > 0)
            def _():
                wait_out(obuf_a, sem_oa)

            compute_group(buf_a, obuf_a)
            start_out(g0, obuf_a, sem_oa)

            wait_group(buf_b, sem_b)

            @pl.when(i < n_groups // 2 - 1)
            def _():
                start_group(g0 + 2, buf_a, sem_a)

            @pl.when(i > 0)
            def _():
                wait_out(obuf_b, sem_ob)

            compute_group(buf_b, obuf_b)
            start_out(g0 + 1, obuf_b, sem_ob)
            return carry

        lax.fori_loop(0, n_groups // 2, pair_loop, 0)
        wait_out(obuf_a, sem_oa)
        wait_out(obuf_b, sem_ob)

    return sc_kernel


def kernel(feature, keypoints):
    B, C, H, W = feature.shape
    N = keypoints.shape[1]
    kpx = keypoints[:, :, 0]
    kpy = keypoints[:, :, 1]
    out = _build_sc_kernel(B, C, H, W, N)(feature, kpx, kpy)
    return out.reshape(B, C, N)
